# baseline (speedup 1.0000x reference)
import jax
import jax.numpy as jnp
from jax import lax
from jax.experimental import pallas as pl
from jax.experimental.pallas import tpu as pltpu
from jax.experimental.pallas import tpu_sc as plsc

B = 4
N = 200000
NXY = 400
C = NXY * NXY
NV = 40000
MP = 32
VOXEL = 0.0025

NSUB = 16
CPT = C // NSUB
ROWS = NV * MP
PROWS = B * ROWS
CROWS = B * NV

VEC = 16
CHUNK = 2000
NCHUNK = N // CHUNK
CPV = CHUNK // VEC
SLICE = 12512
BMWORDS = NSUB * SLICE
CBUF = 1024
CBUF2 = CBUF + VEC
CELLV = CPT // VEC
ZELEM = 4000
BIG = 2**30
WRANKS = 2500
NWIN = NV // WRANKS
WROWS = WRANKS * MP
WELEM = WROWS * 4
LCAP = N
CSLICE = NV // 8


def _body(ptsf_ref, zcon_ref,
          pil_ref, coo2_ref, coo3_ref, npt_ref,
          listd_ref, lv0_ref, lv1_ref, lv2_ref, lv3_ref,
          bitmap, pub, win_sp, coo_sp,
          cnt, first, rank, pbuf, sbuf,
          fidx, fcell, gbuf, onesb,
          cdest, cv0, cv1, cv2, cv3,
          wrel, wv0, wv1, wv2, wv3, gi1,
          zbuf, wstage, cb2, cb3, cnpts, crank, cidx, pubbuf,
          sem0, sem1):
  c = lax.axis_index("c")
  s = lax.axis_index("s")
  iota = lax.iota(jnp.int32, VEC)
  lo = s * CPT
  wid = c * NSUB + s
  coo_dump = CROWS + wid
  bm_dump = N + s * 12

  pltpu.sync_copy(zcon_ref, zbuf)

  def init_buf(ref, val, n=CBUF):
    def w(v, _):
      ref[pl.ds(v * VEC, VEC)] = jnp.zeros((VEC,), jnp.int32) + val
      return 0
    lax.fori_loop(0, n // VEC, w, 0)

  init_buf(onesb, 1)

  def keys_of(v, want_vals):
    i4 = iota * 4 + v * (VEC * 4)
    x = plsc.load_gather(pbuf, [i4])
    y = plsc.load_gather(pbuf, [i4 + 1])
    z = plsc.load_gather(pbuf, [i4 + 2])
    w = plsc.load_gather(pbuf, [i4 + 3]) if want_vals else None
    cx = (x / jnp.float32(VOXEL)).astype(jnp.int32)
    cy = (y / jnp.float32(VOXEL)).astype(jnp.int32)
    cz = z.astype(jnp.int32)
    valid = (cx >= 0) & (cx < NXY) & (cy >= 0) & (cy < NXY) & (cz == 0)
    key = jnp.where(valid, cy * NXY + cx, jnp.int32(BIG))
    return key, x, y, z, w

  def batch_body(half, _):
    b = 2 * half + c
    prow_base = b * ROWS
    crow_base = b * NV

    plsc.subcore_barrier()

    def finit(v, _):
      first[pl.ds(v * VEC, VEC)] = jnp.full((VEC,), BIG, jnp.int32)
      return 0
    lax.fori_loop(0, CELLV, finit, 0, unroll=4)

    def pa_chunk(ch, _):
      pltpu.sync_copy(
          ptsf_ref.at[pl.ds((b * N + ch * CHUNK) * 4, CHUNK * 4)], pbuf)

      def pa_vec(v, _):
        k16, _x, _y, _z, _w = keys_of(v, False)
        m = (k16 >= lo) & (k16 < lo + CPT)
        kl = jnp.where(m, k16 - lo, 0)
        occ, _lm = plsc.scan_count(kl, m)
        f_g = plsc.load_gather(first, [kl], mask=m)
        newm = m & (occ == 1) & (f_g >= BIG)
        pidx = ch * CHUNK + v * VEC + iota
        plsc.store_scatter(first, [kl], pidx, mask=newm)
        return 0

      lax.fori_loop(0, CPV, pa_vec, 0, unroll=2)
      return 0

    lax.fori_loop(0, NCHUNK, pa_chunk, 0)

    def z16(v, _):
      sbuf[pl.ds(v * VEC, VEC)] = jnp.zeros((VEC,), jnp.int32)
      return 0
    lax.fori_loop(0, SLICE // VEC, z16, 0, unroll=4)
    pltpu.sync_copy(sbuf, bitmap.at[pl.ds(s * SLICE, SLICE)])
    plsc.subcore_barrier()

    init_buf(fidx, bm_dump)

    def b2_vec(v, carry):
      off, ecnt = carry
      f16 = first[pl.ds(v * VEC, VEC)]
      m_ne = f16 < BIG
      e01 = jnp.where(m_ne, 0, 1).astype(jnp.int32)
      eincl = plsc.cumsum(e01)
      partial = ecnt + eincl - 1
      rank[pl.ds(v * VEC, VEC)] = jnp.where(m_ne, jnp.int32(BIG), partial)
      ecnt = ecnt + jnp.sum(e01)
      plsc.store_compressed(fidx.at[pl.ds(off, VEC)], f16, mask=m_ne)
      off = off + jnp.sum(jnp.where(m_ne, 1, 0).astype(jnp.int32))
      full = off >= CBUF - VEC

      @pl.when(full)
      def _():
        pltpu.async_copy(onesb, bitmap.at[fidx], sem0).wait()
        init_buf(fidx, bm_dump)

      off = jnp.where(full, 0, off)
      return off, ecnt

    _off, etot = lax.fori_loop(0, CELLV, b2_vec,
                               (jnp.int32(0), jnp.int32(0)))
    pltpu.async_copy(onesb, bitmap.at[fidx], sem0).wait()
    plsc.subcore_barrier()

    pltpu.sync_copy(bitmap.at[pl.ds(s * SLICE, SLICE)], sbuf)

    def b3_vec(v, vacc):
      gidx = s * SLICE + v * VEC + iota
      v16 = sbuf[pl.ds(v * VEC, VEC)]
      return vacc + jnp.where(gidx < N, v16, 0)

    vacc = lax.fori_loop(0, SLICE // VEC, b3_vec,
                         jnp.zeros((VEC,), jnp.int32), unroll=4)
    st = jnp.sum(vacc)
    zv16 = jnp.zeros((VEC,), jnp.int32)
    pubbuf[pl.ds(0, VEC)] = zv16 + st
    pubbuf[pl.ds(VEC, VEC)] = zv16 + etot
    pltpu.sync_copy(pubbuf.at[pl.ds(0, VEC)], pub.at[pl.ds(s * VEC, VEC)])
    pltpu.sync_copy(pubbuf.at[pl.ds(VEC, VEC)],
                    pub.at[pl.ds(NSUB * VEC + s * VEC, VEC)])
    plsc.subcore_barrier()

    pltpu.sync_copy(pub, pubbuf)
    svec = plsc.load_gather(pubbuf, [iota * VEC])
    evec = plsc.load_gather(pubbuf, [NSUB * VEC + iota * VEC])
    t_total = jnp.sum(svec)
    base_s = jnp.sum(jnp.where(iota < s, svec, 0))
    ebase_s = jnp.sum(jnp.where(iota < s, evec, 0))

    def b4_vec(v, carry):
      gidx = s * SLICE + v * VEC + iota
      v16 = jnp.where(gidx < N, sbuf[pl.ds(v * VEC, VEC)], 0)
      incl = plsc.cumsum(v16) + carry
      sbuf[pl.ds(v * VEC, VEC)] = incl
      return carry + jnp.sum(v16)

    lax.fori_loop(0, SLICE // VEC, b4_vec, base_s)
    pltpu.sync_copy(sbuf, bitmap.at[pl.ds(s * SLICE, SLICE)])
    plsc.subcore_barrier()

    def b5_fix(v, _):
      r16 = rank[pl.ds(v * VEC, VEC)]
      m_e = r16 < BIG
      rank[pl.ds(v * VEC, VEC)] = jnp.where(m_e, t_total + ebase_s + r16, r16)
      return 0
    lax.fori_loop(0, CELLV, b5_fix, 0, unroll=2)

    init_buf(fidx, bm_dump)
    init_buf(fcell, CPT)

    def rank_flush():
      pltpu.async_copy(bitmap.at[fidx], gbuf, sem0).wait()

      def w(u, _):
        s16 = gbuf[pl.ds(u * VEC, VEC)]
        c16 = fcell[pl.ds(u * VEC, VEC)]
        plsc.store_scatter(rank, [c16], s16 - 1)
        return 0

      lax.fori_loop(0, CBUF // VEC, w, 0, unroll=2)
      init_buf(fidx, bm_dump)
      init_buf(fcell, CPT)

    def b5_vec(v, off):
      f16 = first[pl.ds(v * VEC, VEC)]
      m_ne = f16 < BIG
      cell16 = v * VEC + iota
      plsc.store_compressed(fidx.at[pl.ds(off, VEC)], f16, mask=m_ne)
      plsc.store_compressed(fcell.at[pl.ds(off, VEC)], cell16, mask=m_ne)
      off = off + jnp.sum(jnp.where(m_ne, 1, 0).astype(jnp.int32))
      full = off >= CBUF - VEC

      @pl.when(full)
      def _():
        rank_flush()

      return jnp.where(full, 0, off)

    lax.fori_loop(0, CELLV, b5_vec, jnp.int32(0))
    rank_flush()

    def c0(v, _):
      cnt[pl.ds(v * VEC, VEC)] = jnp.zeros((VEC,), jnp.int32)
      return 0
    lax.fori_loop(0, CELLV, c0, 0, unroll=4)

    lbase = wid * LCAP

    def list_flush(q):
      dst = pl.ds(lbase + q * CBUF, CBUF)
      pltpu.sync_copy(cdest.at[pl.ds(0, CBUF)], listd_ref.at[dst])
      pltpu.sync_copy(cv0.at[pl.ds(0, CBUF)], lv0_ref.at[dst])
      pltpu.sync_copy(cv1.at[pl.ds(0, CBUF)], lv1_ref.at[dst])
      pltpu.sync_copy(cv2.at[pl.ds(0, CBUF)], lv2_ref.at[dst])
      pltpu.sync_copy(cv3.at[pl.ds(0, CBUF)], lv3_ref.at[dst])
      cdest[pl.ds(0, VEC)] = cdest[pl.ds(CBUF, VEC)]
      cv0[pl.ds(0, VEC)] = cv0[pl.ds(CBUF, VEC)]
      cv1[pl.ds(0, VEC)] = cv1[pl.ds(CBUF, VEC)]
      cv2[pl.ds(0, VEC)] = cv2[pl.ds(CBUF, VEC)]
      cv3[pl.ds(0, VEC)] = cv3[pl.ds(CBUF, VEC)]

    def c2_chunk(ch, carry):
      off, q = carry
      pltpu.sync_copy(
          ptsf_ref.at[pl.ds((b * N + ch * CHUNK) * 4, CHUNK * 4)], pbuf)

      def c2_vec(v, carry):
        off, q = carry
        k16, x, y, z, w = keys_of(v, True)
        m = (k16 >= lo) & (k16 < lo + CPT)
        kl = jnp.where(m, k16 - lo, 0)
        occ, lastm = plsc.scan_count(kl, m)
        cnt_g = plsc.load_gather(cnt, [kl], mask=m)
        plsc.store_scatter(cnt, [kl], cnt_g + occ, mask=m & lastm)
        within = cnt_g + occ - 1
        r_g = plsc.load_gather(rank, [kl], mask=m)
        keep = m & (within < MP) & (r_g < NV)
        dest = prow_base + r_g * MP + within
        dsl = pl.ds(off, VEC)
        plsc.store_compressed(cdest.at[dsl], dest, mask=keep)
        plsc.store_compressed(cv0.at[dsl], x, mask=keep)
        plsc.store_compressed(cv1.at[dsl], y, mask=keep)
        plsc.store_compressed(cv2.at[dsl], z, mask=keep)
        plsc.store_compressed(cv3.at[dsl], w, mask=keep)
        off = off + jnp.sum(jnp.where(keep, 1, 0).astype(jnp.int32))
        full = off >= CBUF

        @pl.when(full)
        def _():
          list_flush(q)

        off = jnp.where(full, off - CBUF, off)
        q = jnp.where(full, q + 1, q)
        return off, q

      return lax.fori_loop(0, CPV, c2_vec, (off, q), unroll=2)

    off, nfull = lax.fori_loop(0, NCHUNK, c2_chunk,
                               (jnp.int32(0), jnp.int32(0)))
    list_flush(nfull)
    llen = nfull * CBUF + off
    nblk = (llen + CBUF - 1) // CBUF
    wslice = WELEM // NSUB

    def wflush():
      for col, wv in ((0, wv0), (1, wv1), (2, wv2), (3, wv3)):
        def tf(u, _, col=col):
          gi1[pl.ds(u * VEC, VEC)] = wrel[pl.ds(u * VEC, VEC)] * 4 + col
          return 0
        lax.fori_loop(0, CBUF // VEC, tf, 0, unroll=4)
        pltpu.async_copy(wv, win_sp.at[gi1], sem0).wait()
      init_buf(wrel, WROWS + s)

    def wpass(w, _):
      wlo = prow_base + w * WROWS

      def wz(zq, _):
        pltpu.sync_copy(
            zbuf, win_sp.at[pl.ds(s * wslice + zq * ZELEM, ZELEM)])
        return 0
      lax.fori_loop(0, wslice // ZELEM, wz, 0)
      plsc.subcore_barrier()

      init_buf(wrel, WROWS + s)

      def c2w_blk(blk, woff):
        src = pl.ds(lbase + blk * CBUF, CBUF)
        pltpu.sync_copy(listd_ref.at[src], cdest.at[pl.ds(0, CBUF)])
        pltpu.sync_copy(lv0_ref.at[src], cv0.at[pl.ds(0, CBUF)])
        pltpu.sync_copy(lv1_ref.at[src], cv1.at[pl.ds(0, CBUF)])
        pltpu.sync_copy(lv2_ref.at[src], cv2.at[pl.ds(0, CBUF)])
        pltpu.sync_copy(lv3_ref.at[src], cv3.at[pl.ds(0, CBUF)])

        def c2w_vec(u, woff):
          d16 = cdest[pl.ds(u * VEC, VEC)]
          eidx = blk * CBUF + u * VEC + iota
          rel = d16 - wlo
          mw = (eidx < llen) & (rel >= 0) & (rel < WROWS)
          dsl = pl.ds(woff, VEC)
          plsc.store_compressed(wrel.at[dsl], rel, mask=mw)
          plsc.store_compressed(wv0.at[dsl], cv0[pl.ds(u * VEC, VEC)],
                                mask=mw)
          plsc.store_compressed(wv1.at[dsl], cv1[pl.ds(u * VEC, VEC)],
                                mask=mw)
          plsc.store_compressed(wv2.at[dsl], cv2[pl.ds(u * VEC, VEC)],
                                mask=mw)
          plsc.store_compressed(wv3.at[dsl], cv3[pl.ds(u * VEC, VEC)],
                                mask=mw)
          woff = woff + jnp.sum(jnp.where(mw, 1, 0).astype(jnp.int32))
          full = woff >= CBUF - VEC

          @pl.when(full)
          def _():
            wflush()

          return jnp.where(full, 0, woff)

        return lax.fori_loop(0, CBUF // VEC, c2w_vec, woff)

      lax.fori_loop(0, nblk, c2w_blk, jnp.int32(0))
      wflush()
      plsc.subcore_barrier()

      def wf(fq, _):
        pltpu.sync_copy(
            win_sp.at[pl.ds(s * wslice + fq * ZELEM, ZELEM)], wstage)
        pltpu.sync_copy(
            wstage,
            pil_ref.at[pl.ds(wlo * 4 + s * wslice + fq * ZELEM, ZELEM)])
        return 0
      lax.fori_loop(0, wslice // ZELEM, wf, 0)
      plsc.subcore_barrier()
      return 0

    lax.fori_loop(0, NWIN, wpass, 0)

    init_buf(crank, 3 * NV + s * VEC, 512)

    def cell_flush():
      d0 = pltpu.async_copy(cb2, coo_sp.at[crank], sem0)
      d0.wait()

      def t2(u, _):
        r = crank[pl.ds(u * VEC, VEC)]
        cidx[pl.ds(u * VEC, VEC)] = r + jnp.where(r < NV, NV, 0)
        return 0
      lax.fori_loop(0, 512 // VEC, t2, 0, unroll=4)
      d1 = pltpu.async_copy(cb3, coo_sp.at[cidx], sem0)
      d1.wait()

      def t3(u, _):
        r = crank[pl.ds(u * VEC, VEC)]
        cidx[pl.ds(u * VEC, VEC)] = r + jnp.where(r < NV, 2 * NV, 0)
        return 0
      lax.fori_loop(0, 512 // VEC, t3, 0, unroll=4)
      d2 = pltpu.async_copy(cnpts, coo_sp.at[cidx], sem1)
      d2.wait()
      init_buf(crank, 3 * NV + s * VEC, 512)

    def c3_vec(v, off):
      r16 = rank[pl.ds(v * VEC, VEC)]
      keep = r16 < NV
      cell16 = lo + v * VEC + iota
      cy = cell16 // NXY
      cx = cell16 % NXY
      cnt16 = cnt[pl.ds(v * VEC, VEC)]
      np16 = jnp.minimum(cnt16, MP)
      k01 = jnp.where(keep, 1, 0).astype(jnp.int32)
      j16 = off + plsc.cumsum(k01) - 1
      plsc.store_scatter(crank, [j16], r16, mask=keep)
      plsc.store_scatter(cnpts, [j16], np16, mask=keep)
      plsc.store_scatter(cb2, [j16], cy, mask=keep)
      plsc.store_scatter(cb3, [j16], cx, mask=keep)
      off = off + jnp.sum(k01)
      full = off >= 512 - VEC

      @pl.when(full)
      def _():
        cell_flush()

      return jnp.where(full, 0, off)

    lax.fori_loop(0, CELLV, c3_vec, jnp.int32(0))
    cell_flush()
    plsc.subcore_barrier()

    @pl.when(s < 8)
    def _():
      for plane, outref in ((0, coo2_ref), (1, coo3_ref), (2, npt_ref)):
        pltpu.sync_copy(coo_sp.at[pl.ds(plane * NV + s * CSLICE, CSLICE)],
                        sbuf.at[pl.ds(0, CSLICE)])
        pltpu.sync_copy(sbuf.at[pl.ds(0, CSLICE)],
                        outref.at[pl.ds(crow_base + s * CSLICE, CSLICE)])
    return 0

  lax.fori_loop(0, 2, batch_body, 0)


@jax.jit
def kernel(batched_pts):
  ptsf = batched_pts.reshape(B * N * 4)
  zcon = jnp.zeros((ZELEM,), jnp.float32)

  mesh = plsc.VectorSubcoreMesh(core_axis_name="c", subcore_axis_name="s")
  run = pl.kernel(
      _body,
      out_type=(
          jax.ShapeDtypeStruct((PROWS * 4 + 128,), jnp.float32),
          jax.ShapeDtypeStruct((CROWS + 32,), jnp.int32),
          jax.ShapeDtypeStruct((CROWS + 32,), jnp.int32),
          jax.ShapeDtypeStruct((CROWS + 32,), jnp.int32),
          jax.ShapeDtypeStruct((2 * NSUB * LCAP,), jnp.int32),
          jax.ShapeDtypeStruct((2 * NSUB * LCAP,), jnp.float32),
          jax.ShapeDtypeStruct((2 * NSUB * LCAP,), jnp.float32),
          jax.ShapeDtypeStruct((2 * NSUB * LCAP,), jnp.float32),
          jax.ShapeDtypeStruct((2 * NSUB * LCAP,), jnp.float32),
      ),
      mesh=mesh,
      compiler_params=pltpu.CompilerParams(needs_layout_passes=False),
      scratch_types=[
          pltpu.VMEM_SHARED((BMWORDS,), jnp.int32),
          pltpu.VMEM_SHARED((2 * NSUB * VEC,), jnp.int32),
          pltpu.VMEM_SHARED((WELEM + 128,), jnp.float32),
          pltpu.VMEM_SHARED((3 * NV + 16 * VEC,), jnp.int32),
          pltpu.VMEM((CPT + VEC,), jnp.int32),
          pltpu.VMEM((CPT + VEC,), jnp.int32),
          pltpu.VMEM((CPT + VEC,), jnp.int32),
          pltpu.VMEM((CHUNK * 4,), jnp.float32),
          pltpu.VMEM((SLICE,), jnp.int32),
          pltpu.VMEM((CBUF,), jnp.int32),
          pltpu.VMEM((CBUF,), jnp.int32),
          pltpu.VMEM((CBUF,), jnp.int32),
          pltpu.VMEM((CBUF,), jnp.int32),
          pltpu.VMEM((CBUF2,), jnp.int32),
          pltpu.VMEM((CBUF2,), jnp.float32),
          pltpu.VMEM((CBUF2,), jnp.float32),
          pltpu.VMEM((CBUF2,), jnp.float32),
          pltpu.VMEM((CBUF2,), jnp.float32),
          pltpu.VMEM((CBUF,), jnp.int32),
          pltpu.VMEM((CBUF,), jnp.float32),
          pltpu.VMEM((CBUF,), jnp.float32),
          pltpu.VMEM((CBUF,), jnp.float32),
          pltpu.VMEM((CBUF,), jnp.float32),
          pltpu.VMEM((CBUF,), jnp.int32),
          pltpu.VMEM((ZELEM,), jnp.float32),
          pltpu.VMEM((ZELEM,), jnp.float32),
          pltpu.VMEM((512,), jnp.int32),
          pltpu.VMEM((512,), jnp.int32),
          pltpu.VMEM((512,), jnp.int32),
          pltpu.VMEM((512,), jnp.int32),
          pltpu.VMEM((512,), jnp.int32),
          pltpu.VMEM((2 * NSUB * VEC,), jnp.int32),
          pltpu.SemaphoreType.DMA,
          pltpu.SemaphoreType.DMA,
      ],
  )
  pil, coo2, coo3, npt, _ld, _l0, _l1, _l2, _l3 = run(ptsf, zcon)
  pillars = pil[:PROWS * 4].reshape(B * NV, MP, 4)
  bcol = jnp.repeat(jnp.arange(B, dtype=jnp.int64), NV)
  coors = jnp.stack(
      [bcol,
       jnp.zeros((CROWS,), jnp.int64),
       coo2[:CROWS].astype(jnp.int64),
       coo3[:CROWS].astype(jnp.int64)], axis=1)
  npts = npt[:CROWS].astype(jnp.int64)
  return pillars, coors, npts


# speedup vs baseline: 1.0046x; 1.0046x over previous
import jax
import jax.numpy as jnp
from jax import lax
from jax.experimental import pallas as pl
from jax.experimental.pallas import tpu as pltpu
from jax.experimental.pallas import tpu_sc as plsc

B = 4
N = 200000
NXY = 400
C = NXY * NXY
NV = 40000
MP = 32
VOXEL = 0.0025

NSUB = 16
CPT = C // NSUB
ROWS = NV * MP
PROWS = B * ROWS
CROWS = B * NV

VEC = 16
CHUNK = 2000
NCHUNK = N // CHUNK
CPV = CHUNK // VEC
SLICE = 12512
BMWORDS = NSUB * SLICE
CBUF = 1024
CBUF2 = CBUF + VEC
CELLV = CPT // VEC
ZELEM = 4000
BIG = 2**30
WRANKS = 2500
NWIN = NV // WRANKS
WROWS = WRANKS * MP
WELEM = WROWS * 4
LCAP = N
CSLICE = NV // 8


def _body(ptsf_ref, zcon_ref,
          pil_ref, coo2_ref, coo3_ref, npt_ref,
          listd_ref, lv0_ref, lv1_ref, lv2_ref, lv3_ref,
          bitmap, pub, win_sp, coo_sp,
          cnt, first, rank, pbuf, sbuf,
          fidx, fcell, gbuf, onesb,
          cdest, cv0, cv1, cv2, cv3,
          wrel, wv0, wv1, wv2, wv3, gi1,
          zbuf, wstage, cb2, cb3, cnpts, crank, cidx, pubbuf,
          sem0, sem1):
  c = lax.axis_index("c")
  s = lax.axis_index("s")
  iota = lax.iota(jnp.int32, VEC)
  lo = s * CPT
  wid = c * NSUB + s
  coo_dump = CROWS + wid
  bm_dump = N + s * 12

  pltpu.sync_copy(zcon_ref, zbuf)

  def init_buf(ref, val, n=CBUF):
    def w(v, _):
      ref[pl.ds(v * VEC, VEC)] = jnp.zeros((VEC,), jnp.int32) + val
      return 0
    lax.fori_loop(0, n // VEC, w, 0)

  init_buf(onesb, 1)

  def keys_of(v, want_vals):
    i4 = iota * 4 + v * (VEC * 4)
    x = plsc.load_gather(pbuf, [i4])
    y = plsc.load_gather(pbuf, [i4 + 1])
    z = plsc.load_gather(pbuf, [i4 + 2])
    w = plsc.load_gather(pbuf, [i4 + 3]) if want_vals else None
    cx = (x / jnp.float32(VOXEL)).astype(jnp.int32)
    cy = (y / jnp.float32(VOXEL)).astype(jnp.int32)
    cz = z.astype(jnp.int32)
    valid = (cx >= 0) & (cx < NXY) & (cy >= 0) & (cy < NXY) & (cz == 0)
    key = jnp.where(valid, cy * NXY + cx, jnp.int32(BIG))
    return key, x, y, z, w

  def batch_body(half, _):
    b = 2 * half + c
    prow_base = b * ROWS
    crow_base = b * NV

    plsc.subcore_barrier()

    def finit(v, _):
      first[pl.ds(v * VEC, VEC)] = jnp.full((VEC,), BIG, jnp.int32)
      return 0
    lax.fori_loop(0, CELLV, finit, 0, unroll=4)

    def pa_chunk(ch, _):
      pltpu.sync_copy(
          ptsf_ref.at[pl.ds((b * N + ch * CHUNK) * 4, CHUNK * 4)], pbuf)

      def pa_vec(v, _):
        k16, _x, _y, _z, _w = keys_of(v, False)
        m = (k16 >= lo) & (k16 < lo + CPT)
        kl = jnp.where(m, k16 - lo, 0)
        occ, _lm = plsc.scan_count(kl, m)
        f_g = plsc.load_gather(first, [kl], mask=m)
        newm = m & (occ == 1) & (f_g >= BIG)
        pidx = ch * CHUNK + v * VEC + iota
        plsc.store_scatter(first, [kl], pidx, mask=newm)
        return 0

      lax.fori_loop(0, CPV, pa_vec, 0, unroll=5)
      return 0

    lax.fori_loop(0, NCHUNK, pa_chunk, 0)

    def z16(v, _):
      sbuf[pl.ds(v * VEC, VEC)] = jnp.zeros((VEC,), jnp.int32)
      return 0
    lax.fori_loop(0, SLICE // VEC, z16, 0, unroll=4)
    pltpu.sync_copy(sbuf, bitmap.at[pl.ds(s * SLICE, SLICE)])
    plsc.subcore_barrier()

    init_buf(fidx, bm_dump)

    def b2_vec(v, carry):
      off, ecnt = carry
      f16 = first[pl.ds(v * VEC, VEC)]
      m_ne = f16 < BIG
      e01 = jnp.where(m_ne, 0, 1).astype(jnp.int32)
      eincl = plsc.cumsum(e01)
      partial = ecnt + eincl - 1
      rank[pl.ds(v * VEC, VEC)] = jnp.where(m_ne, jnp.int32(BIG), partial)
      ecnt = ecnt + jnp.sum(e01)
      plsc.store_compressed(fidx.at[pl.ds(off, VEC)], f16, mask=m_ne)
      off = off + jnp.sum(jnp.where(m_ne, 1, 0).astype(jnp.int32))
      full = off >= CBUF - VEC

      @pl.when(full)
      def _():
        pltpu.async_copy(onesb, bitmap.at[fidx], sem0).wait()
        init_buf(fidx, bm_dump)

      off = jnp.where(full, 0, off)
      return off, ecnt

    _off, etot = lax.fori_loop(0, CELLV, b2_vec,
                               (jnp.int32(0), jnp.int32(0)))
    pltpu.async_copy(onesb, bitmap.at[fidx], sem0).wait()
    plsc.subcore_barrier()

    pltpu.sync_copy(bitmap.at[pl.ds(s * SLICE, SLICE)], sbuf)

    def b3_vec(v, vacc):
      gidx = s * SLICE + v * VEC + iota
      v16 = sbuf[pl.ds(v * VEC, VEC)]
      return vacc + jnp.where(gidx < N, v16, 0)

    vacc = lax.fori_loop(0, SLICE // VEC, b3_vec,
                         jnp.zeros((VEC,), jnp.int32), unroll=4)
    st = jnp.sum(vacc)
    zv16 = jnp.zeros((VEC,), jnp.int32)
    pubbuf[pl.ds(0, VEC)] = zv16 + st
    pubbuf[pl.ds(VEC, VEC)] = zv16 + etot
    pltpu.sync_copy(pubbuf.at[pl.ds(0, VEC)], pub.at[pl.ds(s * VEC, VEC)])
    pltpu.sync_copy(pubbuf.at[pl.ds(VEC, VEC)],
                    pub.at[pl.ds(NSUB * VEC + s * VEC, VEC)])
    plsc.subcore_barrier()

    pltpu.sync_copy(pub, pubbuf)
    svec = plsc.load_gather(pubbuf, [iota * VEC])
    evec = plsc.load_gather(pubbuf, [NSUB * VEC + iota * VEC])
    t_total = jnp.sum(svec)
    base_s = jnp.sum(jnp.where(iota < s, svec, 0))
    ebase_s = jnp.sum(jnp.where(iota < s, evec, 0))

    def b4_vec(v, carry):
      gidx = s * SLICE + v * VEC + iota
      v16 = jnp.where(gidx < N, sbuf[pl.ds(v * VEC, VEC)], 0)
      incl = plsc.cumsum(v16) + carry
      sbuf[pl.ds(v * VEC, VEC)] = incl
      return carry + jnp.sum(v16)

    lax.fori_loop(0, SLICE // VEC, b4_vec, base_s)
    pltpu.sync_copy(sbuf, bitmap.at[pl.ds(s * SLICE, SLICE)])
    plsc.subcore_barrier()

    def b5_fix(v, _):
      r16 = rank[pl.ds(v * VEC, VEC)]
      m_e = r16 < BIG
      rank[pl.ds(v * VEC, VEC)] = jnp.where(m_e, t_total + ebase_s + r16, r16)
      return 0
    lax.fori_loop(0, CELLV, b5_fix, 0, unroll=2)

    init_buf(fidx, bm_dump)
    init_buf(fcell, CPT)

    def rank_flush():
      pltpu.async_copy(bitmap.at[fidx], gbuf, sem0).wait()

      def w(u, _):
        s16 = gbuf[pl.ds(u * VEC, VEC)]
        c16 = fcell[pl.ds(u * VEC, VEC)]
        plsc.store_scatter(rank, [c16], s16 - 1)
        return 0

      lax.fori_loop(0, CBUF // VEC, w, 0, unroll=2)
      init_buf(fidx, bm_dump)
      init_buf(fcell, CPT)

    def b5_vec(v, off):
      f16 = first[pl.ds(v * VEC, VEC)]
      m_ne = f16 < BIG
      cell16 = v * VEC + iota
      plsc.store_compressed(fidx.at[pl.ds(off, VEC)], f16, mask=m_ne)
      plsc.store_compressed(fcell.at[pl.ds(off, VEC)], cell16, mask=m_ne)
      off = off + jnp.sum(jnp.where(m_ne, 1, 0).astype(jnp.int32))
      full = off >= CBUF - VEC

      @pl.when(full)
      def _():
        rank_flush()

      return jnp.where(full, 0, off)

    lax.fori_loop(0, CELLV, b5_vec, jnp.int32(0))
    rank_flush()

    def c0(v, _):
      cnt[pl.ds(v * VEC, VEC)] = jnp.zeros((VEC,), jnp.int32)
      return 0
    lax.fori_loop(0, CELLV, c0, 0, unroll=4)

    lbase = wid * LCAP

    def list_flush(q):
      dst = pl.ds(lbase + q * CBUF, CBUF)
      pltpu.sync_copy(cdest.at[pl.ds(0, CBUF)], listd_ref.at[dst])
      pltpu.sync_copy(cv0.at[pl.ds(0, CBUF)], lv0_ref.at[dst])
      pltpu.sync_copy(cv1.at[pl.ds(0, CBUF)], lv1_ref.at[dst])
      pltpu.sync_copy(cv2.at[pl.ds(0, CBUF)], lv2_ref.at[dst])
      pltpu.sync_copy(cv3.at[pl.ds(0, CBUF)], lv3_ref.at[dst])
      cdest[pl.ds(0, VEC)] = cdest[pl.ds(CBUF, VEC)]
      cv0[pl.ds(0, VEC)] = cv0[pl.ds(CBUF, VEC)]
      cv1[pl.ds(0, VEC)] = cv1[pl.ds(CBUF, VEC)]
      cv2[pl.ds(0, VEC)] = cv2[pl.ds(CBUF, VEC)]
      cv3[pl.ds(0, VEC)] = cv3[pl.ds(CBUF, VEC)]

    def c2_chunk(ch, carry):
      off, q = carry
      pltpu.sync_copy(
          ptsf_ref.at[pl.ds((b * N + ch * CHUNK) * 4, CHUNK * 4)], pbuf)

      def c2_vec(v, carry):
        off, q = carry
        k16, x, y, z, w = keys_of(v, True)
        m = (k16 >= lo) & (k16 < lo + CPT)
        kl = jnp.where(m, k16 - lo, 0)
        occ, lastm = plsc.scan_count(kl, m)
        cnt_g = plsc.load_gather(cnt, [kl], mask=m)
        plsc.store_scatter(cnt, [kl], cnt_g + occ, mask=m & lastm)
        within = cnt_g + occ - 1
        r_g = plsc.load_gather(rank, [kl], mask=m)
        keep = m & (within < MP) & (r_g < NV)
        dest = prow_base + r_g * MP + within
        dsl = pl.ds(off, VEC)
        plsc.store_compressed(cdest.at[dsl], dest, mask=keep)
        plsc.store_compressed(cv0.at[dsl], x, mask=keep)
        plsc.store_compressed(cv1.at[dsl], y, mask=keep)
        plsc.store_compressed(cv2.at[dsl], z, mask=keep)
        plsc.store_compressed(cv3.at[dsl], w, mask=keep)
        off = off + jnp.sum(jnp.where(keep, 1, 0).astype(jnp.int32))
        full = off >= CBUF

        @pl.when(full)
        def _():
          list_flush(q)

        off = jnp.where(full, off - CBUF, off)
        q = jnp.where(full, q + 1, q)
        return off, q

      return lax.fori_loop(0, CPV, c2_vec, (off, q), unroll=5)

    off, nfull = lax.fori_loop(0, NCHUNK, c2_chunk,
                               (jnp.int32(0), jnp.int32(0)))
    list_flush(nfull)
    llen = nfull * CBUF + off
    nblk = (llen + CBUF - 1) // CBUF
    wslice = WELEM // NSUB

    def wflush():
      for col, wv in ((0, wv0), (1, wv1), (2, wv2), (3, wv3)):
        def tf(u, _, col=col):
          gi1[pl.ds(u * VEC, VEC)] = wrel[pl.ds(u * VEC, VEC)] * 4 + col
          return 0
        lax.fori_loop(0, CBUF // VEC, tf, 0, unroll=4)
        pltpu.async_copy(wv, win_sp.at[gi1], sem0).wait()
      init_buf(wrel, WROWS + s)

    def wpass(w, _):
      wlo = prow_base + w * WROWS

      def wz(zq, _):
        pltpu.sync_copy(
            zbuf, win_sp.at[pl.ds(s * wslice + zq * ZELEM, ZELEM)])
        return 0
      lax.fori_loop(0, wslice // ZELEM, wz, 0)
      plsc.subcore_barrier()

      init_buf(wrel, WROWS + s)

      def c2w_blk(blk, woff):
        src = pl.ds(lbase + blk * CBUF, CBUF)
        pltpu.sync_copy(listd_ref.at[src], cdest.at[pl.ds(0, CBUF)])
        pltpu.sync_copy(lv0_ref.at[src], cv0.at[pl.ds(0, CBUF)])
        pltpu.sync_copy(lv1_ref.at[src], cv1.at[pl.ds(0, CBUF)])
        pltpu.sync_copy(lv2_ref.at[src], cv2.at[pl.ds(0, CBUF)])
        pltpu.sync_copy(lv3_ref.at[src], cv3.at[pl.ds(0, CBUF)])

        def c2w_vec(u, woff):
          d16 = cdest[pl.ds(u * VEC, VEC)]
          eidx = blk * CBUF + u * VEC + iota
          rel = d16 - wlo
          mw = (eidx < llen) & (rel >= 0) & (rel < WROWS)
          dsl = pl.ds(woff, VEC)
          plsc.store_compressed(wrel.at[dsl], rel, mask=mw)
          plsc.store_compressed(wv0.at[dsl], cv0[pl.ds(u * VEC, VEC)],
                                mask=mw)
          plsc.store_compressed(wv1.at[dsl], cv1[pl.ds(u * VEC, VEC)],
                                mask=mw)
          plsc.store_compressed(wv2.at[dsl], cv2[pl.ds(u * VEC, VEC)],
                                mask=mw)
          plsc.store_compressed(wv3.at[dsl], cv3[pl.ds(u * VEC, VEC)],
                                mask=mw)
          woff = woff + jnp.sum(jnp.where(mw, 1, 0).astype(jnp.int32))
          full = woff >= CBUF - VEC

          @pl.when(full)
          def _():
            wflush()

          return jnp.where(full, 0, woff)

        return lax.fori_loop(0, CBUF // VEC, c2w_vec, woff)

      lax.fori_loop(0, nblk, c2w_blk, jnp.int32(0))
      wflush()
      plsc.subcore_barrier()

      def wf(fq, _):
        pltpu.sync_copy(
            win_sp.at[pl.ds(s * wslice + fq * ZELEM, ZELEM)], wstage)
        pltpu.sync_copy(
            wstage,
            pil_ref.at[pl.ds(wlo * 4 + s * wslice + fq * ZELEM, ZELEM)])
        return 0
      lax.fori_loop(0, wslice // ZELEM, wf, 0)
      plsc.subcore_barrier()
      return 0

    lax.fori_loop(0, NWIN, wpass, 0)

    init_buf(crank, 3 * NV + s * VEC, 512)

    def cell_flush():
      d0 = pltpu.async_copy(cb2, coo_sp.at[crank], sem0)
      d0.wait()

      def t2(u, _):
        r = crank[pl.ds(u * VEC, VEC)]
        cidx[pl.ds(u * VEC, VEC)] = r + jnp.where(r < NV, NV, 0)
        return 0
      lax.fori_loop(0, 512 // VEC, t2, 0, unroll=4)
      d1 = pltpu.async_copy(cb3, coo_sp.at[cidx], sem0)
      d1.wait()

      def t3(u, _):
        r = crank[pl.ds(u * VEC, VEC)]
        cidx[pl.ds(u * VEC, VEC)] = r + jnp.where(r < NV, 2 * NV, 0)
        return 0
      lax.fori_loop(0, 512 // VEC, t3, 0, unroll=4)
      d2 = pltpu.async_copy(cnpts, coo_sp.at[cidx], sem1)
      d2.wait()
      init_buf(crank, 3 * NV + s * VEC, 512)

    def c3_vec(v, off):
      r16 = rank[pl.ds(v * VEC, VEC)]
      keep = r16 < NV
      cell16 = lo + v * VEC + iota
      cy = cell16 // NXY
      cx = cell16 % NXY
      cnt16 = cnt[pl.ds(v * VEC, VEC)]
      np16 = jnp.minimum(cnt16, MP)
      k01 = jnp.where(keep, 1, 0).astype(jnp.int32)
      j16 = off + plsc.cumsum(k01) - 1
      plsc.store_scatter(crank, [j16], r16, mask=keep)
      plsc.store_scatter(cnpts, [j16], np16, mask=keep)
      plsc.store_scatter(cb2, [j16], cy, mask=keep)
      plsc.store_scatter(cb3, [j16], cx, mask=keep)
      off = off + jnp.sum(k01)
      full = off >= 512 - VEC

      @pl.when(full)
      def _():
        cell_flush()

      return jnp.where(full, 0, off)

    lax.fori_loop(0, CELLV, c3_vec, jnp.int32(0))
    cell_flush()
    plsc.subcore_barrier()

    @pl.when(s < 8)
    def _():
      for plane, outref in ((0, coo2_ref), (1, coo3_ref), (2, npt_ref)):
        pltpu.sync_copy(coo_sp.at[pl.ds(plane * NV + s * CSLICE, CSLICE)],
                        sbuf.at[pl.ds(0, CSLICE)])
        pltpu.sync_copy(sbuf.at[pl.ds(0, CSLICE)],
                        outref.at[pl.ds(crow_base + s * CSLICE, CSLICE)])
    return 0

  lax.fori_loop(0, 2, batch_body, 0)


@jax.jit
def kernel(batched_pts):
  ptsf = batched_pts.reshape(B * N * 4)
  zcon = jnp.zeros((ZELEM,), jnp.float32)

  mesh = plsc.VectorSubcoreMesh(core_axis_name="c", subcore_axis_name="s")
  run = pl.kernel(
      _body,
      out_type=(
          jax.ShapeDtypeStruct((PROWS * 4 + 128,), jnp.float32),
          jax.ShapeDtypeStruct((CROWS + 32,), jnp.int32),
          jax.ShapeDtypeStruct((CROWS + 32,), jnp.int32),
          jax.ShapeDtypeStruct((CROWS + 32,), jnp.int32),
          jax.ShapeDtypeStruct((2 * NSUB * LCAP,), jnp.int32),
          jax.ShapeDtypeStruct((2 * NSUB * LCAP,), jnp.float32),
          jax.ShapeDtypeStruct((2 * NSUB * LCAP,), jnp.float32),
          jax.ShapeDtypeStruct((2 * NSUB * LCAP,), jnp.float32),
          jax.ShapeDtypeStruct((2 * NSUB * LCAP,), jnp.float32),
      ),
      mesh=mesh,
      compiler_params=pltpu.CompilerParams(needs_layout_passes=False),
      scratch_types=[
          pltpu.VMEM_SHARED((BMWORDS,), jnp.int32),
          pltpu.VMEM_SHARED((2 * NSUB * VEC,), jnp.int32),
          pltpu.VMEM_SHARED((WELEM + 128,), jnp.float32),
          pltpu.VMEM_SHARED((3 * NV + 16 * VEC,), jnp.int32),
          pltpu.VMEM((CPT + VEC,), jnp.int32),
          pltpu.VMEM((CPT + VEC,), jnp.int32),
          pltpu.VMEM((CPT + VEC,), jnp.int32),
          pltpu.VMEM((CHUNK * 4,), jnp.float32),
          pltpu.VMEM((SLICE,), jnp.int32),
          pltpu.VMEM((CBUF,), jnp.int32),
          pltpu.VMEM((CBUF,), jnp.int32),
          pltpu.VMEM((CBUF,), jnp.int32),
          pltpu.VMEM((CBUF,), jnp.int32),
          pltpu.VMEM((CBUF2,), jnp.int32),
          pltpu.VMEM((CBUF2,), jnp.float32),
          pltpu.VMEM((CBUF2,), jnp.float32),
          pltpu.VMEM((CBUF2,), jnp.float32),
          pltpu.VMEM((CBUF2,), jnp.float32),
          pltpu.VMEM((CBUF,), jnp.int32),
          pltpu.VMEM((CBUF,), jnp.float32),
          pltpu.VMEM((CBUF,), jnp.float32),
          pltpu.VMEM((CBUF,), jnp.float32),
          pltpu.VMEM((CBUF,), jnp.float32),
          pltpu.VMEM((CBUF,), jnp.int32),
          pltpu.VMEM((ZELEM,), jnp.float32),
          pltpu.VMEM((ZELEM,), jnp.float32),
          pltpu.VMEM((512,), jnp.int32),
          pltpu.VMEM((512,), jnp.int32),
          pltpu.VMEM((512,), jnp.int32),
          pltpu.VMEM((512,), jnp.int32),
          pltpu.VMEM((512,), jnp.int32),
          pltpu.VMEM((2 * NSUB * VEC,), jnp.int32),
          pltpu.SemaphoreType.DMA,
          pltpu.SemaphoreType.DMA,
      ],
  )
  pil, coo2, coo3, npt, _ld, _l0, _l1, _l2, _l3 = run(ptsf, zcon)
  pillars = pil[:PROWS * 4].reshape(B * NV, MP, 4)
  bcol = jnp.repeat(jnp.arange(B, dtype=jnp.int64), NV)
  coors = jnp.stack(
      [bcol,
       jnp.zeros((CROWS,), jnp.int64),
       coo2[:CROWS].astype(jnp.int64),
       coo3[:CROWS].astype(jnp.int64)], axis=1)
  npts = npt[:CROWS].astype(jnp.int64)
  return pillars, coors, npts
